# Initial kernel scaffold; baseline (speedup 1.0000x reference)
import jax
import jax.numpy as jnp
from jax import lax
from jax.experimental import pallas as pl
from jax.experimental.pallas import tpu as pltpu
from jax.experimental.pallas import tpu_sc as plsc

B = 1024
S = 50
E = 128
H = 256
B2 = 2 * B
FEAT = 2 * E + 4 * H
NW = 32
HW = NW // 2
ROWS = B * S
RPW = ROWS // HW
CH = 128
GRP = 5
GROUPS = RPW // (CH * GRP)
NCHUNK = RPW // CH


def _sc_gather_body(vac_tab, res_tab, vac_idx, res_idx, vac_out, res_out,
                    idx_v, rows_v, sem):
    cid = lax.axis_index("c")
    sid = lax.axis_index("s")
    wid = sid * 2 + cid
    hw = lax.rem(wid, HW)
    base = hw * RPW

    def run(tab, idx_hbm, out_hbm):
        pltpu.sync_copy(idx_hbm.at[hw], idx_v)
        for g in range(GROUPS):
            copies = [
                pltpu.async_copy(tab.at[idx_v.at[g * GRP + j]],
                                 rows_v.at[pl.ds(j * CH, CH)], sem)
                for j in range(GRP)
            ]
            for c in copies:
                c.wait()
            pltpu.sync_copy(rows_v,
                            out_hbm.at[pl.ds(base + g * (GRP * CH), GRP * CH)])

    @pl.when(wid < HW)
    def _():
        run(vac_tab, vac_idx, vac_out)

    @pl.when(wid >= HW)
    def _():
        run(res_tab, res_idx, res_out)


_sc_gather = pl.kernel(
    _sc_gather_body,
    out_type=(
        jax.ShapeDtypeStruct((ROWS, E), jnp.float32),
        jax.ShapeDtypeStruct((ROWS, E), jnp.float32),
    ),
    mesh=plsc.VectorSubcoreMesh(core_axis_name="c", subcore_axis_name="s"),
    scratch_types=[
        pltpu.VMEM((NCHUNK, CH), jnp.int32),
        pltpu.VMEM((GRP * CH, E), jnp.float32),
        pltpu.SemaphoreType.DMA,
    ],
)


def _lstm_body(xv_ref, xr_ref, wih_ref, whh_ref, b_ref, w1_ref, b1_ref,
               w2_ref, b2_ref, out_ref,
               h_s, c_s, rmax_s, rsum_s, emax_s, esum_s, cat_s):
    t = pl.program_id(0)
    x = jnp.concatenate([xv_ref[0], xr_ref[0]], axis=0)

    @pl.when(t == 0)
    def _():
        h_s[...] = jnp.zeros((B2, H), jnp.float32)
        c_s[...] = jnp.zeros((B2, H), jnp.float32)
        rmax_s[...] = jnp.full((B2, H), -jnp.inf, jnp.float32)
        rsum_s[...] = jnp.zeros((B2, H), jnp.float32)
        emax_s[...] = jnp.full((B2, E), -jnp.inf, jnp.float32)
        esum_s[...] = jnp.zeros((B2, E), jnp.float32)

    h = h_s[...]
    c = c_s[...]
    gates = (jnp.dot(x, wih_ref[...], preferred_element_type=jnp.float32)
             + jnp.dot(h, whh_ref[...], preferred_element_type=jnp.float32)
             + b_ref[...])
    gi = jax.nn.sigmoid(gates[:, 0:H])
    gf = jax.nn.sigmoid(gates[:, H:2 * H])
    gg = jnp.tanh(gates[:, 2 * H:3 * H])
    go = jax.nn.sigmoid(gates[:, 3 * H:4 * H])
    cn = gf * c + gi * gg
    hn = go * jnp.tanh(cn)
    h_s[...] = hn
    c_s[...] = cn
    rmax_s[...] = jnp.maximum(rmax_s[...], hn)
    rsum_s[...] = rsum_s[...] + hn
    emax_s[...] = jnp.maximum(emax_s[...], x)
    esum_s[...] = esum_s[...] + x

    @pl.when(t == S - 1)
    def _():
        inv = jnp.float32(1.0 / B)
        emax = emax_s[...]
        esum = esum_s[...] * inv
        rmax = rmax_s[...]
        rsum = rsum_s[...] * inv
        hT = h_s[...]
        cT = c_s[...]
        for half in range(2):
            off = half * FEAT
            r0, r1 = half * B, (half + 1) * B
            cat_s[:, off + 0:off + E] = emax[r0:r1, :]
            cat_s[:, off + E:off + 2 * E] = esum[r0:r1, :]
            cat_s[:, off + 2 * E:off + 2 * E + H] = rmax[r0:r1, :]
            cat_s[:, off + 2 * E + H:off + 2 * E + 2 * H] = rsum[r0:r1, :]
            cat_s[:, off + 2 * E + 2 * H:off + 2 * E + 3 * H] = hT[r0:r1, :]
            cat_s[:, off + 2 * E + 3 * H:off + 2 * E + 4 * H] = cT[r0:r1, :]
        cat = cat_s[...]
        h1 = jnp.maximum(
            jnp.dot(cat, w1_ref[...], preferred_element_type=jnp.float32)
            + b1_ref[...], 0.0)
        out_ref[...] = jax.nn.sigmoid(
            jnp.dot(h1, w2_ref[...], preferred_element_type=jnp.float32)
            + b2_ref[...])


_tc_lstm = pl.pallas_call(
    _lstm_body,
    grid=(S,),
    in_specs=[
        pl.BlockSpec((1, B, E), lambda t: (t, 0, 0)),
        pl.BlockSpec((1, B, E), lambda t: (t, 0, 0)),
        pl.BlockSpec((E, 4 * H), lambda t: (0, 0)),
        pl.BlockSpec((H, 4 * H), lambda t: (0, 0)),
        pl.BlockSpec((1, 4 * H), lambda t: (0, 0)),
        pl.BlockSpec((2 * FEAT, 512), lambda t: (0, 0)),
        pl.BlockSpec((1, 512), lambda t: (0, 0)),
        pl.BlockSpec((512, 128), lambda t: (0, 0)),
        pl.BlockSpec((1, 128), lambda t: (0, 0)),
    ],
    out_specs=pl.BlockSpec((B, 128), lambda t: (0, 0)),
    out_shape=jax.ShapeDtypeStruct((B, 128), jnp.float32),
    scratch_shapes=[
        pltpu.VMEM((B2, H), jnp.float32),
        pltpu.VMEM((B2, H), jnp.float32),
        pltpu.VMEM((B2, H), jnp.float32),
        pltpu.VMEM((B2, H), jnp.float32),
        pltpu.VMEM((B2, E), jnp.float32),
        pltpu.VMEM((B2, E), jnp.float32),
        pltpu.VMEM((B, 2 * FEAT), jnp.float32),
    ],
    compiler_params=pltpu.CompilerParams(dimension_semantics=("arbitrary",)),
)


def kernel(vac_text, res_text, vac_table, res_table, W_ih, W_hh, b_ih, b_hh,
           fc1_W, fc1_b, fc2_W, fc2_b):
    vac_idx = vac_text.astype(jnp.int32).T.reshape(HW, NCHUNK, CH)
    res_idx = res_text.astype(jnp.int32).T.reshape(HW, NCHUNK, CH)
    vac_e, res_e = _sc_gather(vac_table, res_table, vac_idx, res_idx)
    xv = vac_e.reshape(S, B, E)
    xr = res_e.reshape(S, B, E)
    bias = (b_ih + b_hh).reshape(1, 4 * H)
    return _tc_lstm(xv, xr, W_ih.T, W_hh.T, bias, fc1_W.T,
                    fc1_b.reshape(1, -1), fc2_W.T, fc2_b.reshape(1, -1))


# speedup vs baseline: 5.3576x; 5.3576x over previous
import functools

import jax
import jax.numpy as jnp
from jax import lax
from jax.experimental import pallas as pl
from jax.experimental.pallas import tpu as pltpu
from jax.experimental.pallas import tpu_sc as plsc

B = 1024
S = 50
E = 128
H = 256
B2 = 2 * B
FEAT = 2 * E + 4 * H
NW = 32
ROWS = B * S
RPW = ROWS // NW
CH = 64
GRP = 5
GROUPS = RPW // (CH * GRP)
NCHUNK = RPW // CH


def _sc_gather_body(vac_tab, res_tab, vac_idx, res_idx, vac_out, res_out,
                    idx_v, rows_v, sem):
    cid = lax.axis_index("c")
    sid = lax.axis_index("s")
    wid = sid * 2 + cid
    base = wid * RPW

    def run(tab, idx_hbm, out_hbm):
        pltpu.sync_copy(idx_hbm.at[wid], idx_v)
        for g in range(GROUPS):
            copies = [
                pltpu.async_copy(tab.at[idx_v.at[g * GRP + j]],
                                 rows_v.at[pl.ds(j * CH, CH)], sem)
                for j in range(GRP)
            ]
            for c in copies:
                c.wait()
            pltpu.sync_copy(rows_v,
                            out_hbm.at[pl.ds(base + g * (GRP * CH), GRP * CH)])

    run(vac_tab, vac_idx, vac_out)
    run(res_tab, res_idx, res_out)


@functools.cache
def _sc_gather():
    return pl.kernel(
        _sc_gather_body,
        out_type=(
            jax.ShapeDtypeStruct((ROWS, E), jnp.float32),
            jax.ShapeDtypeStruct((ROWS, E), jnp.float32),
        ),
        mesh=plsc.VectorSubcoreMesh(core_axis_name="c", subcore_axis_name="s"),
        scratch_types=[
            pltpu.VMEM((NCHUNK, CH), jnp.int32),
            pltpu.VMEM((GRP * CH, E), jnp.float32),
            pltpu.SemaphoreType.DMA,
        ],
    )


def _lstm_body(xv_ref, xr_ref, wih_ref, whh_ref, b_ref, w1_ref, b1_ref,
               w2_ref, b2_ref, out_ref,
               h_s, c_s, rmax_s, rsum_s, emax_s, esum_s, cat_s):
    t = pl.program_id(0)
    x = jnp.concatenate([xv_ref[0], xr_ref[0]], axis=0)

    @pl.when(t == 0)
    def _():
        h_s[...] = jnp.zeros((B2, H), jnp.float32)
        c_s[...] = jnp.zeros((B2, H), jnp.float32)
        rmax_s[...] = jnp.full((B2, H), -jnp.inf, jnp.float32)
        rsum_s[...] = jnp.zeros((B2, H), jnp.float32)
        emax_s[...] = jnp.full((B2, E), -jnp.inf, jnp.float32)
        esum_s[...] = jnp.zeros((B2, E), jnp.float32)

    h = h_s[...]
    c = c_s[...]
    gates = (jnp.dot(x, wih_ref[...], preferred_element_type=jnp.float32)
             + jnp.dot(h, whh_ref[...], preferred_element_type=jnp.float32)
             + b_ref[...])
    gi = jax.nn.sigmoid(gates[:, 0:H])
    gf = jax.nn.sigmoid(gates[:, H:2 * H])
    gg = jnp.tanh(gates[:, 2 * H:3 * H])
    go = jax.nn.sigmoid(gates[:, 3 * H:4 * H])
    cn = gf * c + gi * gg
    hn = go * jnp.tanh(cn)
    h_s[...] = hn
    c_s[...] = cn
    rmax_s[...] = jnp.maximum(rmax_s[...], hn)
    rsum_s[...] = rsum_s[...] + hn
    emax_s[...] = jnp.maximum(emax_s[...], x)
    esum_s[...] = esum_s[...] + x

    @pl.when(t == S - 1)
    def _():
        inv = jnp.float32(1.0 / B)
        emax = emax_s[...]
        esum = esum_s[...] * inv
        rmax = rmax_s[...]
        rsum = rsum_s[...] * inv
        hT = h_s[...]
        cT = c_s[...]
        for half in range(2):
            off = half * FEAT
            r0, r1 = half * B, (half + 1) * B
            cat_s[:, off + 0:off + E] = emax[r0:r1, :]
            cat_s[:, off + E:off + 2 * E] = esum[r0:r1, :]
            cat_s[:, off + 2 * E:off + 2 * E + H] = rmax[r0:r1, :]
            cat_s[:, off + 2 * E + H:off + 2 * E + 2 * H] = rsum[r0:r1, :]
            cat_s[:, off + 2 * E + 2 * H:off + 2 * E + 3 * H] = hT[r0:r1, :]
            cat_s[:, off + 2 * E + 3 * H:off + 2 * E + 4 * H] = cT[r0:r1, :]
        cat = cat_s[...]
        h1 = jnp.maximum(
            jnp.dot(cat, w1_ref[...], preferred_element_type=jnp.float32)
            + b1_ref[...], 0.0)
        out_ref[...] = jax.nn.sigmoid(
            jnp.dot(h1, w2_ref[...], preferred_element_type=jnp.float32)
            + b2_ref[...])


_tc_lstm = pl.pallas_call(
    _lstm_body,
    grid=(S,),
    in_specs=[
        pl.BlockSpec((1, B, E), lambda t: (t, 0, 0)),
        pl.BlockSpec((1, B, E), lambda t: (t, 0, 0)),
        pl.BlockSpec((E, 4 * H), lambda t: (0, 0)),
        pl.BlockSpec((H, 4 * H), lambda t: (0, 0)),
        pl.BlockSpec((1, 4 * H), lambda t: (0, 0)),
        pl.BlockSpec((2 * FEAT, 512), lambda t: (0, 0)),
        pl.BlockSpec((1, 512), lambda t: (0, 0)),
        pl.BlockSpec((512, 128), lambda t: (0, 0)),
        pl.BlockSpec((1, 128), lambda t: (0, 0)),
    ],
    out_specs=pl.BlockSpec((B, 128), lambda t: (0, 0)),
    out_shape=jax.ShapeDtypeStruct((B, 128), jnp.float32),
    scratch_shapes=[
        pltpu.VMEM((B2, H), jnp.float32),
        pltpu.VMEM((B2, H), jnp.float32),
        pltpu.VMEM((B2, H), jnp.float32),
        pltpu.VMEM((B2, H), jnp.float32),
        pltpu.VMEM((B2, E), jnp.float32),
        pltpu.VMEM((B2, E), jnp.float32),
        pltpu.VMEM((B, 2 * FEAT), jnp.float32),
    ],
    compiler_params=pltpu.CompilerParams(dimension_semantics=("arbitrary",)),
)


def kernel(vac_text, res_text, vac_table, res_table, W_ih, W_hh, b_ih, b_hh,
           fc1_W, fc1_b, fc2_W, fc2_b):
    vac_idx = vac_text.astype(jnp.int32).T.reshape(NW, NCHUNK, CH)
    res_idx = res_text.astype(jnp.int32).T.reshape(NW, NCHUNK, CH)
    vac_e, res_e = _sc_gather()(vac_table, res_table, vac_idx, res_idx)
    xv = vac_e.reshape(S, B, E)
    xr = res_e.reshape(S, B, E)
    bias = (b_ih + b_hh).reshape(1, 4 * H)
    return _tc_lstm(xv, xr, W_ih.T, W_hh.T, bias, fc1_W.T,
                    fc1_b.reshape(1, -1), fc2_W.T, fc2_b.reshape(1, -1))
